# Initial kernel scaffold; baseline (speedup 1.0000x reference)
import jax
import jax.numpy as jnp
from jax.experimental import pallas as pl


def kernel(x, edge_index, edge_attr, batch, params):
    raise NotImplementedError("write your pallas kernel here")


# speedup vs baseline: 2.6667x; 2.6667x over previous
import dataclasses
import functools

import jax
import jax.numpy as jnp
from jax import lax
from jax.experimental import pallas as pl
from jax.experimental.pallas import tpu as pltpu
from jax.experimental.pallas import tpu_sc as plsc

N = 10000
E = 320000
D = 128
DE = 16
L = 3
G = 64
STEPS = 5

NC = 2
NS = 16
LANES = 16
CHUNK = 128
CHUNKS_PER_TILE = 79
EP = NC * NS * CHUNKS_PER_TILE * CHUNK
N2 = 10240
ZROWS = 128
RPT = N2 // NS


def _leaky(x):
    return jnp.where(x >= 0, x, 0.01 * x)


def _edge_mlp_body(ea_ref, we1_ref, be1_ref, we2_ref, be2_ref, out_ref):
    a = ea_ref[...]
    cols = []
    for i in range(L):
        t = jnp.dot(a, we1_ref[i], preferred_element_type=jnp.float32) + be1_ref[i]
        t = _leaky(t)
        u = jnp.dot(t, we2_ref[i], preferred_element_type=jnp.float32) + be2_ref[i]
        cols.append(jnp.where(u >= 0, u, jnp.exp(u) - 1.0))
    out_ref[...] = jnp.concatenate(cols, axis=1)


def _edge_weights(edge_attr, we1s, be1s, we2s, be2s):
    be = 3200
    grid = E // be
    return pl.pallas_call(
        _edge_mlp_body,
        grid=(grid,),
        in_specs=[
            pl.BlockSpec((be, DE), lambda i: (i, 0)),
            pl.BlockSpec((L, DE, 8), lambda i: (0, 0, 0)),
            pl.BlockSpec((L, 1, 8), lambda i: (0, 0, 0)),
            pl.BlockSpec((L, 8, 1), lambda i: (0, 0, 0)),
            pl.BlockSpec((L, 1, 1), lambda i: (0, 0, 0)),
        ],
        out_specs=pl.BlockSpec((be, L), lambda i: (i, 0)),
        out_shape=jax.ShapeDtypeStruct((E, L), jnp.float32),
    )(edge_attr, we1s, be1s, we2s, be2s)


@functools.cache
def _sc_segment_kernel():
    mesh = plsc.VectorSubcoreMesh(core_axis_name="c", subcore_axis_name="s")
    cp = pltpu.CompilerParams()
    if "needs_layout_passes" in pltpu.CompilerParams.__dataclass_fields__:
        cp = dataclasses.replace(cp, needs_layout_passes=False)

    @functools.partial(
        pl.kernel,
        compiler_params=cp,
        out_type=jax.ShapeDtypeStruct((NC * N2, D), jnp.float32),
        mesh=mesh,
        scratch_types=[
            pltpu.VMEM((CHUNK,), jnp.int32),
            pltpu.VMEM((CHUNK,), jnp.int32),
            pltpu.VMEM((CHUNK,), jnp.float32),
            pltpu.VMEM((CHUNK, D), jnp.float32),
            pltpu.VMEM((ZROWS, D), jnp.float32),
            pltpu.VMEM_SHARED((N2, D), jnp.float32),
            pltpu.SemaphoreType.DMA,
        ],
    )
    def sc_kernel(h_hbm, src_hbm, dst_hbm, w_hbm, out_hbm,
                  srcb, dstb, wb, rows, zb, shared, sem):
        c = lax.axis_index("c")
        s = lax.axis_index("s")
        wid = c * NS + s

        @pl.loop(0, ZROWS)
        def _zero(r):
            for j in range(D // LANES):
                zb[r, pl.ds(j * LANES, LANES)] = jnp.zeros((LANES,), jnp.float32)

        for k in range(RPT // ZROWS):
            pltpu.sync_copy(zb, shared.at[pl.ds(s * RPT + k * ZROWS, ZROWS)])
        plsc.subcore_barrier()

        base = wid * (CHUNKS_PER_TILE * CHUNK)

        @pl.loop(0, CHUNKS_PER_TILE)
        def _edges(t):
            e0 = base + t * CHUNK
            pltpu.sync_copy(src_hbm.at[pl.ds(e0, CHUNK)], srcb)
            pltpu.sync_copy(dst_hbm.at[pl.ds(e0, CHUNK)], dstb)
            pltpu.sync_copy(w_hbm.at[pl.ds(e0, CHUNK)], wb)
            pltpu.async_copy(h_hbm.at[srcb], rows, sem).wait()

            @pl.loop(0, CHUNK)
            def _scale(e):
                wv = plsc.load_gather(wb, [jnp.full((LANES,), e, jnp.int32)])
                for j in range(D // LANES):
                    sl = pl.ds(j * LANES, LANES)
                    rows[e, sl] = rows[e, sl] * wv

            pltpu.sync_copy(rows, shared.at[dstb], add=True)

        plsc.subcore_barrier()
        for k in range(RPT // ZROWS):
            r0 = s * RPT + k * ZROWS
            pltpu.sync_copy(shared.at[pl.ds(r0, ZROWS)],
                            out_hbm.at[pl.ds(c * N2 + r0, ZROWS)])

    return sc_kernel


def _sc_segment_sum(h, src_p, dst_p, w_p):
    part = _sc_segment_kernel()(h, src_p, dst_p, w_p)
    return part.reshape(NC, N2, D)[:, :N]


_RC = 1000


def _gin_mm_body(part_ref, h_ref, w1_ref, b1_ref, w2_ref, b2_ref, eps_ref,
                 t_ref, sum_ref, sq_ref):
    j = pl.program_id(0)
    z = part_ref[0] + part_ref[1] + (1.0 + eps_ref[0, 0]) * h_ref[...]
    t = jnp.dot(z, w1_ref[...], preferred_element_type=jnp.float32) + b1_ref[...]
    t = _leaky(t)
    t = jnp.dot(t, w2_ref[...], preferred_element_type=jnp.float32) + b2_ref[...]
    t = _leaky(t)
    t_ref[...] = t
    colsum = jnp.sum(t, axis=0, keepdims=True)
    colsq = jnp.sum(t * t, axis=0, keepdims=True)

    @pl.when(j == 0)
    def _():
        sum_ref[...] = colsum
        sq_ref[...] = colsq

    @pl.when(j > 0)
    def _():
        sum_ref[...] += colsum
        sq_ref[...] += colsq


def _gin_mm(part, h, w1, b1, w2, b2, eps):
    grid = N // _RC
    return pl.pallas_call(
        _gin_mm_body,
        grid=(grid,),
        in_specs=[
            pl.BlockSpec((NC, _RC, D), lambda j: (0, j, 0)),
            pl.BlockSpec((_RC, D), lambda j: (j, 0)),
            pl.BlockSpec((D, D), lambda j: (0, 0)),
            pl.BlockSpec((1, D), lambda j: (0, 0)),
            pl.BlockSpec((D, D), lambda j: (0, 0)),
            pl.BlockSpec((1, D), lambda j: (0, 0)),
            pl.BlockSpec((1, 1), lambda j: (0, 0)),
        ],
        out_specs=[
            pl.BlockSpec((_RC, D), lambda j: (j, 0)),
            pl.BlockSpec((1, D), lambda j: (0, 0)),
            pl.BlockSpec((1, D), lambda j: (0, 0)),
        ],
        out_shape=[
            jax.ShapeDtypeStruct((N, D), jnp.float32),
            jax.ShapeDtypeStruct((1, D), jnp.float32),
            jax.ShapeDtypeStruct((1, D), jnp.float32),
        ],
    )(part, h, w1, b1, w2, b2, eps)


def _bn_body(t_ref, sum_ref, sq_ref, g_ref, bt_ref, out_ref):
    mu = sum_ref[...] * (1.0 / N)
    var = sq_ref[...] * (1.0 / N) - mu * mu
    inv = lax.rsqrt(var + 1e-5)
    out_ref[...] = (t_ref[...] - mu) * inv * g_ref[...] + bt_ref[...]


def _bn(t, colsum, colsq, gamma, beta):
    grid = N // _RC
    return pl.pallas_call(
        _bn_body,
        grid=(grid,),
        in_specs=[
            pl.BlockSpec((_RC, D), lambda j: (j, 0)),
            pl.BlockSpec((1, D), lambda j: (0, 0)),
            pl.BlockSpec((1, D), lambda j: (0, 0)),
            pl.BlockSpec((1, D), lambda j: (0, 0)),
            pl.BlockSpec((1, D), lambda j: (0, 0)),
        ],
        out_specs=pl.BlockSpec((_RC, D), lambda j: (j, 0)),
        out_shape=jax.ShapeDtypeStruct((N, D), jnp.float32),
    )(t, colsum, colsq, gamma, beta)


def _bn_max_body(t_ref, sum_ref, sq_ref, g_ref, bt_ref, h1_ref, h2_ref, out_ref):
    mu = sum_ref[...] * (1.0 / N)
    var = sq_ref[...] * (1.0 / N) - mu * mu
    inv = lax.rsqrt(var + 1e-5)
    h3 = (t_ref[...] - mu) * inv * g_ref[...] + bt_ref[...]
    out_ref[...] = jnp.maximum(jnp.maximum(h1_ref[...], h2_ref[...]), h3)


def _bn_max(t, colsum, colsq, gamma, beta, h1, h2):
    grid = N // _RC
    return pl.pallas_call(
        _bn_max_body,
        grid=(grid,),
        in_specs=[
            pl.BlockSpec((_RC, D), lambda j: (j, 0)),
            pl.BlockSpec((1, D), lambda j: (0, 0)),
            pl.BlockSpec((1, D), lambda j: (0, 0)),
            pl.BlockSpec((1, D), lambda j: (0, 0)),
            pl.BlockSpec((1, D), lambda j: (0, 0)),
            pl.BlockSpec((_RC, D), lambda j: (j, 0)),
            pl.BlockSpec((_RC, D), lambda j: (j, 0)),
        ],
        out_specs=pl.BlockSpec((_RC, D), lambda j: (j, 0)),
        out_shape=jax.ShapeDtypeStruct((N, D), jnp.float32),
    )(t, colsum, colsq, gamma, beta, h1, h2)


_RD = 1000
_NB = N // _RD
_TD = STEPS * _NB + 1


def _rowscale(v, m):
    eye = lax.broadcasted_iota(jnp.int32, (G, G), 0) == lax.broadcasted_iota(
        jnp.int32, (G, G), 1)
    dg = jnp.where(eye, jnp.broadcast_to(v, (G, G)), 0.0)
    return jnp.dot(dg, m, preferred_element_type=jnp.float32)


def _set2set_body(hx_ref, batch_ref,
                  wih0_ref, whh0_ref, bih0_ref, bhh0_ref,
                  wih1_ref, whh1_ref, bih1_ref, bhh1_ref,
                  out_ref,
                  qs_ref, q_ref, h0_ref, c0_ref, h1_ref, c1_ref,
                  m_ref, s_ref, r_ref):
    t = pl.program_id(0)
    new_step = (t % _NB == 0) & (t < STEPS * _NB)

    def r_norm():
        srun = s_ref[...]
        recip = jnp.where(srun > 0.5, 1.0 / jnp.maximum(srun, 0.5), 0.0)
        return _rowscale(recip, r_ref[...])

    @pl.when(t == 0)
    def _():
        qs_ref[...] = jnp.zeros((G, 2 * D), jnp.float32)
        h0_ref[...] = jnp.zeros((G, D), jnp.float32)
        c0_ref[...] = jnp.zeros((G, D), jnp.float32)
        h1_ref[...] = jnp.zeros((G, D), jnp.float32)
        c1_ref[...] = jnp.zeros((G, D), jnp.float32)

    @pl.when((t > 0) & (t % _NB == 0))
    def _():
        qs_ref[...] = jnp.concatenate([q_ref[...], r_norm()], axis=1)

    @pl.when(new_step)
    def _():
        inp = qs_ref[...]
        hs = [h0_ref, c0_ref, h1_ref, c1_ref]
        for (wih, whh, bih, bhh, h_r, c_r) in (
            (wih0_ref, whh0_ref, bih0_ref, bhh0_ref, h0_ref, c0_ref),
            (wih1_ref, whh1_ref, bih1_ref, bhh1_ref, h1_ref, c1_ref),
        ):
            gates = (jnp.dot(inp, wih[...], preferred_element_type=jnp.float32)
                     + bih[...]
                     + jnp.dot(h_r[...], whh[...], preferred_element_type=jnp.float32)
                     + bhh[...])
            ig = jax.nn.sigmoid(gates[:, 0:D])
            fg = jax.nn.sigmoid(gates[:, D:2 * D])
            gg = jnp.tanh(gates[:, 2 * D:3 * D])
            og = jax.nn.sigmoid(gates[:, 3 * D:4 * D])
            cnew = fg * c_r[...] + ig * gg
            hnew = og * jnp.tanh(cnew)
            c_r[...] = cnew
            h_r[...] = hnew
            inp = hnew
        q_ref[...] = inp
        m_ref[...] = jnp.full((1, G), -1e30, jnp.float32)
        s_ref[...] = jnp.zeros((1, G), jnp.float32)
        r_ref[...] = jnp.zeros((G, D), jnp.float32)

    @pl.when(t < STEPS * _NB)
    def _():
        hx = hx_ref[...]
        onehot = batch_ref[...] == lax.broadcasted_iota(jnp.int32, (_RD, G), 1)
        onef = jnp.where(onehot, 1.0, 0.0)
        qb = jnp.dot(onef, q_ref[...], preferred_element_type=jnp.float32)
        e = jnp.sum(hx * qb, axis=1, keepdims=True)
        m_blk = jnp.max(jnp.where(onehot, e, -1e30), axis=0, keepdims=True)
        m_old = m_ref[...]
        m_new = jnp.maximum(m_old, m_blk)
        scale = jnp.exp(m_old - m_new)
        mb = jnp.sum(onef * m_new, axis=1, keepdims=True)
        ex_oh = onef * jnp.exp(e - mb)
        s_ref[...] = s_ref[...] * scale + jnp.sum(ex_oh, axis=0, keepdims=True)
        r_ref[...] = _rowscale(scale, r_ref[...]) + lax.dot_general(
            ex_oh, hx, (((0,), (0,)), ((), ())),
            preferred_element_type=jnp.float32)
        m_ref[...] = m_new

    @pl.when(t == _TD - 1)
    def _():
        out_ref[...] = jnp.concatenate([q_ref[...], r_norm()], axis=1)


def _set2set(hx, batch2d, lstm_t):
    (wih0, whh0, bih0, bhh0, wih1, whh1, bih1, bhh1) = lstm_t
    return pl.pallas_call(
        _set2set_body,
        grid=(_TD,),
        in_specs=[
            pl.BlockSpec((_RD, D), lambda t: (t % _NB, 0)),
            pl.BlockSpec((_RD, 1), lambda t: (t % _NB, 0)),
            pl.BlockSpec((2 * D, 4 * D), lambda t: (0, 0)),
            pl.BlockSpec((D, 4 * D), lambda t: (0, 0)),
            pl.BlockSpec((1, 4 * D), lambda t: (0, 0)),
            pl.BlockSpec((1, 4 * D), lambda t: (0, 0)),
            pl.BlockSpec((D, 4 * D), lambda t: (0, 0)),
            pl.BlockSpec((D, 4 * D), lambda t: (0, 0)),
            pl.BlockSpec((1, 4 * D), lambda t: (0, 0)),
            pl.BlockSpec((1, 4 * D), lambda t: (0, 0)),
        ],
        out_specs=pl.BlockSpec((G, 2 * D), lambda t: (0, 0)),
        out_shape=jax.ShapeDtypeStruct((G, 2 * D), jnp.float32),
        scratch_shapes=[
            pltpu.VMEM((G, 2 * D), jnp.float32),
            pltpu.VMEM((G, D), jnp.float32),
            pltpu.VMEM((G, D), jnp.float32),
            pltpu.VMEM((G, D), jnp.float32),
            pltpu.VMEM((G, D), jnp.float32),
            pltpu.VMEM((G, D), jnp.float32),
            pltpu.VMEM((1, G), jnp.float32),
            pltpu.VMEM((1, G), jnp.float32),
            pltpu.VMEM((G, D), jnp.float32),
        ],
    )(hx, batch2d, wih0, whh0, bih0, bhh0, wih1, whh1, bih1, bhh1)


def kernel(x, edge_index, edge_attr, batch, params):
    convs = params["convs"]
    bns = params["bns"]
    lstm = params["lstm"]

    we1s = jnp.stack([p["We1"] for p in convs])
    be1s = jnp.stack([p["be1"] for p in convs])[:, None, :]
    we2s = jnp.stack([p["We2"] for p in convs])
    be2s = jnp.stack([p["be2"] for p in convs])[:, :, None]

    w_all = _edge_weights(edge_attr, we1s, be1s, we2s, be2s)
    w_pad = jnp.pad(w_all.T, ((0, 0), (0, EP - E)))
    src_p = jnp.pad(edge_index[0], (0, EP - E))
    dst_p = jnp.pad(edge_index[1], (0, EP - E))

    h = x
    layer_h = []
    for i in range(L):
        p = convs[i]
        part = _sc_segment_sum(h, src_p, dst_p, w_pad[i])
        t, colsum, colsq = _gin_mm(
            part, h, p["W1"], p["b1"][None, :], p["W2"], p["b2"][None, :],
            p["eps"][None, None])
        bn = bns[i]
        if i < L - 1:
            h = _bn(t, colsum, colsq, bn["gamma"][None, :], bn["beta"][None, :])
            layer_h.append(h)
        else:
            hx = _bn_max(t, colsum, colsq, bn["gamma"][None, :],
                         bn["beta"][None, :], layer_h[0], layer_h[1])

    lstm_t = (
        lstm[0]["Wih"].T, lstm[0]["Whh"].T,
        lstm[0]["bih"][None, :], lstm[0]["bhh"][None, :],
        lstm[1]["Wih"].T, lstm[1]["Whh"].T,
        lstm[1]["bih"][None, :], lstm[1]["bhh"][None, :],
    )
    return _set2set(hx, batch[:, None], lstm_t)
